# baseline (speedup 1.0000x reference)
import functools

import jax
import jax.numpy as jnp
from jax import lax
from jax.experimental import pallas as pl
from jax.experimental.pallas import tpu as pltpu
from jax.experimental.pallas import tpu_sc as plsc

F32 = jnp.float32
BF16 = jnp.bfloat16

_NC = 2
_NS = 16
_NW = _NC * _NS
_CHUNK = 128


def _pack_bf16_pair(a, b):
    au = lax.bitcast_convert_type(a.astype(BF16), jnp.uint16)
    bu = lax.bitcast_convert_type(b.astype(BF16), jnp.uint16)
    word = au.astype(jnp.uint32) | (bu.astype(jnp.uint32) << 16)
    return lax.bitcast_convert_type(word, F32)


def _unpack_bf16_pair(w):
    u = lax.bitcast_convert_type(w, jnp.uint32)
    a = lax.bitcast_convert_type((u & 0xFFFF).astype(jnp.uint16), BF16)
    b = lax.bitcast_convert_type((u >> 16).astype(jnp.uint16), BF16)
    return a, b


def _mm_body(x_ref, w_ref, o_ref):
    y = jnp.dot(x_ref[...], w_ref[...], preferred_element_type=F32)
    if o_ref.shape[1] * 2 == y.shape[1]:
        parts = []
        for gi in range(y.shape[1] // 256):
            parts.append(_pack_bf16_pair(y[:, gi * 256:gi * 256 + 128],
                                         y[:, gi * 256 + 128:(gi + 1) * 256]))
        y = jnp.concatenate(parts, axis=-1) if len(parts) > 1 else parts[0]
    o_ref[...] = y.astype(o_ref.dtype)


def _tc_matmul(x, w, block_m, pack=False):
    m, k = x.shape
    _, n = w.shape
    n_out = n // 2 if pack else n
    grid = m // block_m
    return pl.pallas_call(
        _mm_body,
        grid=(grid,),
        in_specs=[
            pl.BlockSpec((block_m, k), lambda i: (i, 0)),
            pl.BlockSpec((k, n), lambda i: (0, 0)),
        ],
        out_specs=pl.BlockSpec((block_m, n_out), lambda i: (i, 0)),
        out_shape=jax.ShapeDtypeStruct((m, n_out), F32),
    )(x, w)


def _edge_mlp_body(g_ref, e_ref, w1e_ref, b1_ref, w2_ref, b2_ref, o_ref):
    a0, b0 = _unpack_bf16_pair(g_ref[0, ...])
    a1, b1h = _unpack_bf16_pair(g_ref[1, ...])
    gsum = (jnp.concatenate([a0, b0], axis=-1).astype(F32)
            + jnp.concatenate([a1, b1h], axis=-1).astype(F32))
    h = (gsum
         + jnp.dot(e_ref[...], w1e_ref[...], preferred_element_type=F32)
         + b1_ref[...])
    h = jnp.maximum(h, 0.0).astype(BF16)
    msg = jnp.dot(h, w2_ref[...], preferred_element_type=F32) + b2_ref[...]
    msg = jnp.maximum(msg, 0.0)
    o_ref[0, ...] = msg[:, :128]
    o_ref[1, ...] = msg[:, 128:]


def _tc_edge_mlp(g, e, w1e, b1, w2, b2, block_e):
    _, n_edges, gpack = g.shape
    emb = w2.shape[0]
    edim = e.shape[1]
    grid = n_edges // block_e
    return pl.pallas_call(
        _edge_mlp_body,
        grid=(grid,),
        in_specs=[
            pl.BlockSpec((2, block_e, gpack), lambda i: (0, i, 0)),
            pl.BlockSpec((block_e, edim), lambda i: (i, 0)),
            pl.BlockSpec((edim, emb), lambda i: (0, 0)),
            pl.BlockSpec((1, emb), lambda i: (0, 0)),
            pl.BlockSpec((emb, emb), lambda i: (0, 0)),
            pl.BlockSpec((1, emb), lambda i: (0, 0)),
        ],
        out_specs=pl.BlockSpec((2, block_e, 128), lambda i: (0, i, 0)),
        out_shape=jax.ShapeDtypeStruct((2, n_edges, 128), F32),
    )(g, e, w1e, b1, w2, b2)


def _node_mlp_body(x_ref, *refs):
    (w1x_ref, w1a_ref, b1_ref, w2_ref, b2_ref, wp_ref, o_ref, p_ref) = \
        refs[-8:]
    a_refs = refs[:-8]
    a0 = sum(a[0, ...] for a in a_refs)
    a1 = sum(a[1, ...] for a in a_refs)
    agg = jnp.concatenate([a0, a1], axis=-1).astype(BF16)
    t = (jnp.dot(x_ref[...].astype(BF16), w1x_ref[...],
                 preferred_element_type=F32)
         + jnp.dot(agg, w1a_ref[...], preferred_element_type=F32)
         + b1_ref[...])
    t = jnp.maximum(t, 0.0).astype(BF16)
    y = jnp.dot(t, w2_ref[...], preferred_element_type=F32) + b2_ref[...]
    y = jnp.maximum(y, 0.0)
    o_ref[...] = y
    p = jnp.dot(y.astype(BF16), wp_ref[...], preferred_element_type=F32)
    p_ref[...] = _pack_bf16_pair(p[:, :128], p[:, 128:])


def _tc_node_mlp(x, aggs, w1x, w1a, b1, w2, b2, wp, block_m):
    n, emb = x.shape
    grid = n // block_m
    return pl.pallas_call(
        _node_mlp_body,
        grid=(grid,),
        in_specs=[
            pl.BlockSpec((block_m, emb), lambda i: (i, 0)),
        ] + [
            pl.BlockSpec((2, block_m, 128), lambda i: (0, i, 0))
            for _ in aggs
        ] + [
            pl.BlockSpec((emb, emb), lambda i: (0, 0)),
            pl.BlockSpec((emb, emb), lambda i: (0, 0)),
            pl.BlockSpec((1, emb), lambda i: (0, 0)),
            pl.BlockSpec((emb, emb), lambda i: (0, 0)),
            pl.BlockSpec((1, emb), lambda i: (0, 0)),
            pl.BlockSpec((emb, emb), lambda i: (0, 0)),
        ],
        out_specs=[
            pl.BlockSpec((block_m, emb), lambda i: (i, 0)),
            pl.BlockSpec((block_m, emb // 2), lambda i: (i, 0)),
        ],
        out_shape=[
            jax.ShapeDtypeStruct((n, emb), F32),
            jax.ShapeDtypeStruct((n, emb // 2), F32),
        ],
    )(x, *aggs, w1x, w1a, b1, w2, b2, wp)


def _sc_gather_body(n_edges, pc_hbm, pv_hbm, row_hbm, col_hbm, g_hbm,
                    idxr_all, idxc_all, gb0, gb1, tb0, tb1,
                    sg0, sg1, st0, st1, so0, so1, sp0, sp1):
    cid = lax.axis_index("c")
    sid = lax.axis_index("s")
    w = sid * _NC + cid
    epw = n_edges // _NW
    nfull = epw // _CHUNK
    tail = epw - nfull * _CHUNK
    ebase = w * epw

    pltpu.sync_copy(row_hbm.at[pl.ds(ebase, epw)], idxr_all)
    pltpu.sync_copy(col_hbm.at[pl.ds(ebase, epw)], idxc_all)

    gbufs = (gb0, gb1)
    tbufs = (tb0, tb1)
    sgs = (sg0, sg1)
    sts = (st0, st1)
    sos = (so0, so1)
    sps = (sp0, sp1)

    def start_gathers(k, bo):
        idx_r = idxr_all.at[pl.ds(k * _CHUNK, _CHUNK)]
        idx_c = idxc_all.at[pl.ds(k * _CHUNK, _CHUNK)]
        pltpu.async_copy(pc_hbm.at[idx_r], gbufs[bo], sgs[bo])
        pltpu.async_copy(pv_hbm.at[idx_c], tbufs[bo], sts[bo])

    def wait_gathers(bo):
        pltpu.make_async_copy(
            pc_hbm.at[idxr_all.at[pl.ds(0, _CHUNK)]], gbufs[bo],
            sgs[bo]).wait()
        pltpu.make_async_copy(
            pv_hbm.at[idxc_all.at[pl.ds(0, _CHUNK)]], tbufs[bo],
            sts[bo]).wait()

    def wait_writes(bo):
        pltpu.make_async_copy(
            gbufs[bo], g_hbm.at[0, pl.ds(0, _CHUNK)], sos[bo]).wait()
        pltpu.make_async_copy(
            tbufs[bo], g_hbm.at[1, pl.ds(0, _CHUNK)], sps[bo]).wait()

    start_gathers(0, 0)

    @pl.loop(1, 2 * ((nfull + 2) // 2) + 1, step=2)
    def _(g):
        for bo in range(2):
            k = g + bo
            b = bo ^ 1

            @pl.when(k <= nfull)
            def _():
                @pl.when((k >= 2) & (k < nfull))
                def _():
                    wait_writes(b)

                @pl.when(k < nfull)
                def _():
                    start_gathers(k, b)
                wait_gathers(b ^ 1)
                base = ebase + (k - 1) * _CHUNK
                pltpu.async_copy(gbufs[b ^ 1],
                                 g_hbm.at[0, pl.ds(base, _CHUNK)], sos[b ^ 1])
                pltpu.async_copy(tbufs[b ^ 1],
                                 g_hbm.at[1, pl.ds(base, _CHUNK)], sps[b ^ 1])

    for bo in range(min(2, nfull)):
        wait_writes(bo)

    if tail:
        toff = nfull * _CHUNK
        it_r = idxr_all.at[pl.ds(toff, tail)]
        it_c = idxc_all.at[pl.ds(toff, tail)]
        pltpu.async_copy(pc_hbm.at[it_r], gb0.at[pl.ds(0, tail)], sg0).wait()
        pltpu.async_copy(pv_hbm.at[it_c], tb0.at[pl.ds(0, tail)], st0).wait()
        pltpu.sync_copy(gb0.at[pl.ds(0, tail)],
                        g_hbm.at[0, pl.ds(ebase + toff, tail)])
        pltpu.sync_copy(tb0.at[pl.ds(0, tail)],
                        g_hbm.at[1, pl.ds(ebase + toff, tail)])


def _sc_gather_combine(pc, pv, row, col):
    n_edges = row.shape[0]
    emb = pc.shape[1]
    dt = pc.dtype
    epw = n_edges // _NW
    mesh = plsc.VectorSubcoreMesh(core_axis_name="c", subcore_axis_name="s")
    return pl.kernel(
        functools.partial(_sc_gather_body, n_edges),
        out_type=jax.ShapeDtypeStruct((2, n_edges, emb), dt),
        mesh=mesh,
        scratch_types=[
            pltpu.VMEM((epw,), jnp.int32),
            pltpu.VMEM((epw,), jnp.int32),
            pltpu.VMEM((_CHUNK, emb), dt),
            pltpu.VMEM((_CHUNK, emb), dt),
            pltpu.VMEM((_CHUNK, emb), dt),
            pltpu.VMEM((_CHUNK, emb), dt),
        ] + [pltpu.SemaphoreType.DMA] * 8,
    )(pc, pv, row, col)


def _sc_scatter_body(n_edges, rows_per_tile, msg_hbm, idx_hbm, zero_hbm,
                     agg_hbm, mb0, mb1, ic0, ic1, itail, acc_sh,
                     sm0, sm1, si0, si1):
    cid = lax.axis_index("c")
    sid = lax.axis_index("s")
    rbase = sid * rows_per_tile

    n_zc = rows_per_tile // _CHUNK
    pltpu.sync_copy(zero_hbm, mb0)
    for z in range(n_zc):
        pltpu.sync_copy(mb0, acc_sh.at[pl.ds(rbase + z * _CHUNK, _CHUNK)])
    plsc.subcore_barrier()

    epw = n_edges // _NS
    nfull = epw // _CHUNK
    tail = epw - nfull * _CHUNK
    ebase = sid * epw

    mbs = (mb0, mb1)
    sms = (sm0, sm1)
    icur = (ic0, ic1)
    sis = (si0, si1)
    for bo in range(2):
        pltpu.async_copy(msg_hbm.at[cid, pl.ds(ebase + bo * _CHUNK, _CHUNK)],
                         mbs[bo], sms[bo])
        pltpu.async_copy(idx_hbm.at[pl.ds(ebase + bo * _CHUNK, _CHUNK)],
                         icur[bo], sis[bo])

    @pl.loop(0, nfull, step=2)
    def _(g):
        for bo in range(2):
            k = g + bo
            pltpu.make_async_copy(
                msg_hbm.at[cid, pl.ds(0, _CHUNK)], mbs[bo], sms[bo]).wait()
            pltpu.make_async_copy(
                idx_hbm.at[pl.ds(0, _CHUNK)], icur[bo], sis[bo]).wait()
            pltpu.sync_copy(mbs[bo], acc_sh.at[icur[bo]], add=True)

            @pl.when(k + 2 < nfull)
            def _():
                nb = ebase + (k + 2) * _CHUNK
                pltpu.async_copy(
                    msg_hbm.at[cid, pl.ds(nb, _CHUNK)], mbs[bo], sms[bo])
                pltpu.async_copy(
                    idx_hbm.at[pl.ds(nb, _CHUNK)], icur[bo], sis[bo])

    if tail:
        toff = nfull * _CHUNK
        cm = pltpu.async_copy(msg_hbm.at[cid, pl.ds(ebase + toff, tail)],
                              mb0.at[pl.ds(0, tail)], sm0)
        ci = pltpu.async_copy(idx_hbm.at[pl.ds(ebase + toff, tail)],
                              itail, si0)
        cm.wait()
        ci.wait()
        pltpu.sync_copy(mb0.at[pl.ds(0, tail)], acc_sh.at[itail], add=True)
    plsc.subcore_barrier()

    for z in range(n_zc):
        pltpu.sync_copy(acc_sh.at[pl.ds(rbase + z * _CHUNK, _CHUNK)], mb0)
        pltpu.sync_copy(mb0, agg_hbm.at[cid, pl.ds(rbase + z * _CHUNK, _CHUNK)])


def _sc_scatter_add(msg2, idx, n_nodes):
    n_edges = idx.shape[0]
    half = msg2.shape[2]
    rows_per_tile = (-(-n_nodes // _NS) + _CHUNK - 1) // _CHUNK * _CHUNK
    n_pad = rows_per_tile * _NS
    zero = jnp.zeros((_CHUNK, half), F32)
    mesh = plsc.VectorSubcoreMesh(core_axis_name="c", subcore_axis_name="s")
    epw = n_edges // _NS
    tail = epw - (epw // _CHUNK) * _CHUNK
    return pl.kernel(
        functools.partial(_sc_scatter_body, n_edges, rows_per_tile),
        out_type=jax.ShapeDtypeStruct((2, n_pad, half), F32),
        mesh=mesh,
        scratch_types=[
            pltpu.VMEM((_CHUNK, half), F32),
            pltpu.VMEM((_CHUNK, half), F32),
            pltpu.VMEM((_CHUNK,), jnp.int32),
            pltpu.VMEM((_CHUNK,), jnp.int32),
            pltpu.VMEM((tail or _CHUNK,), jnp.int32),
            pltpu.VMEM_SHARED((n_pad, half), F32),
            pltpu.SemaphoreType.DMA,
            pltpu.SemaphoreType.DMA,
            pltpu.SemaphoreType.DMA,
            pltpu.SemaphoreType.DMA,
        ],
    )(msg2, idx, zero)


def kernel(c, v, edge_index, e,
           gC_W1, gC_b1, gC_W2, gC_b2,
           gV_W1, gV_b1, gV_W2, gV_b2,
           fC_W1, fC_b1, fC_W2, fC_b2,
           fV_W1, fV_b1, fV_W2, fV_b2,
           beta_c, sigma_c, beta_v, sigma_v):
    n_c, emb = c.shape
    n_v = v.shape[0]
    row = edge_index[0]
    col = edge_index[1]

    gC_W1c, gC_W1v, gC_W1e = gC_W1[:emb], gC_W1[emb:2 * emb], gC_W1[2 * emb:]
    gV_W1c, gV_W1v, gV_W1e = gV_W1[:emb], gV_W1[emb:2 * emb], gV_W1[2 * emb:]

    fC_W1x, fC_W1a = fC_W1[:emb], fC_W1[emb:]
    fV_W1x, fV_W1a = fV_W1[:emb], fV_W1[emb:]
    fC_W1a_eff = fC_W1a / sigma_c[:, None]
    fC_b1_eff = fC_b1 - (beta_c / sigma_c) @ fC_W1a
    fV_W1a_eff = fV_W1a / sigma_v[:, None]
    fV_b1_eff = fV_b1 - (beta_v / sigma_v) @ fV_W1a

    b2 = lambda x: x.reshape(1, -1)
    bf = lambda x: x.astype(BF16)

    pc1 = _tc_matmul(c, gC_W1c, 5000, pack=True)
    pv_both = _tc_matmul(v, jnp.concatenate([gC_W1v, gV_W1v], axis=1), 5000,
                         pack=True)
    pv1 = pv_both[:, :emb // 2]
    pv2 = pv_both[:, emb // 2:]

    n_edges = row.shape[0]
    quantum = 32 * _CHUNK
    n_parts = 3
    cut = [min((n_edges * i // n_parts + quantum - 1) // quantum * quantum,
               n_edges) for i in range(n_parts + 1)]
    halves = [(row[a:b], col[a:b], e[a:b])
              for a, b in zip(cut[:-1], cut[1:]) if b > a]

    def phase(pc, pv, gW1e, gb1, gW2, gb2, dst, nn):
        msgs = []
        for rh, ch, eh in halves:
            gh = _sc_gather_combine(pc, pv, rh, ch)
            bl = rh.shape[0] // 32
            msgs.append(_tc_edge_mlp(gh, eh, gW1e, b2(gb1), bf(gW2),
                                     b2(gb2), bl))
        aggs = [_sc_scatter_add(m, (rh if dst == 0 else ch), nn)
                for m, (rh, ch, _) in zip(msgs, halves)]
        return aggs

    agg_c = phase(pc1, pv1, gC_W1e, gC_b1, gC_W2, gC_b2, 0, n_c)
    c_new, pc2 = _tc_node_mlp(c, agg_c, bf(fC_W1x),
                              bf(fC_W1a_eff), b2(fC_b1_eff), bf(fC_W2),
                              b2(fC_b2), bf(gV_W1c), 5000)

    agg_v = phase(pc2, pv2, gV_W1e, gV_b1, gV_W2, gV_b2, 1, n_v)
    v_new, _ = _tc_node_mlp(v, agg_v, bf(fV_W1x),
                            bf(fV_W1a_eff), b2(fV_b1_eff), bf(fV_W2),
                            b2(fV_b2), bf(fV_W2), 5000)

    return (c_new, v_new)


# speedup vs baseline: 1.0374x; 1.0374x over previous
import functools

import jax
import jax.numpy as jnp
from jax import lax
from jax.experimental import pallas as pl
from jax.experimental.pallas import tpu as pltpu
from jax.experimental.pallas import tpu_sc as plsc

F32 = jnp.float32
BF16 = jnp.bfloat16

_NC = 2
_NS = 16
_NW = _NC * _NS
_CHUNK = 128


def _pack_bf16_pair(a, b):
    au = lax.bitcast_convert_type(a.astype(BF16), jnp.uint16)
    bu = lax.bitcast_convert_type(b.astype(BF16), jnp.uint16)
    word = au.astype(jnp.uint32) | (bu.astype(jnp.uint32) << 16)
    return lax.bitcast_convert_type(word, F32)


def _unpack_bf16_pair(w):
    u = lax.bitcast_convert_type(w, jnp.uint32)
    a = lax.bitcast_convert_type((u & 0xFFFF).astype(jnp.uint16), BF16)
    b = lax.bitcast_convert_type((u >> 16).astype(jnp.uint16), BF16)
    return a, b


def _mm_body(x_ref, w_ref, o_ref):
    y = jnp.dot(x_ref[...], w_ref[...], preferred_element_type=F32)
    if o_ref.shape[1] * 2 == y.shape[1]:
        parts = []
        for gi in range(y.shape[1] // 256):
            parts.append(_pack_bf16_pair(y[:, gi * 256:gi * 256 + 128],
                                         y[:, gi * 256 + 128:(gi + 1) * 256]))
        y = jnp.concatenate(parts, axis=-1) if len(parts) > 1 else parts[0]
    o_ref[...] = y.astype(o_ref.dtype)


def _tc_matmul(x, w, block_m, pack=False):
    m, k = x.shape
    _, n = w.shape
    n_out = n // 2 if pack else n
    grid = m // block_m
    return pl.pallas_call(
        _mm_body,
        grid=(grid,),
        in_specs=[
            pl.BlockSpec((block_m, k), lambda i: (i, 0)),
            pl.BlockSpec((k, n), lambda i: (0, 0)),
        ],
        out_specs=pl.BlockSpec((block_m, n_out), lambda i: (i, 0)),
        out_shape=jax.ShapeDtypeStruct((m, n_out), F32),
    )(x, w)


def _edge_mlp_body(g_ref, e_ref, w1e_ref, b1_ref, w2_ref, b2_ref, o_ref):
    a0, b0 = _unpack_bf16_pair(g_ref[0, ...])
    a1, b1h = _unpack_bf16_pair(g_ref[1, ...])
    gsum = (jnp.concatenate([a0, b0], axis=-1).astype(F32)
            + jnp.concatenate([a1, b1h], axis=-1).astype(F32))
    h = (gsum
         + jnp.dot(e_ref[...], w1e_ref[...], preferred_element_type=F32)
         + b1_ref[...])
    h = jnp.maximum(h, 0.0).astype(BF16)
    msg = jnp.dot(h, w2_ref[...], preferred_element_type=F32) + b2_ref[...]
    msg = jnp.maximum(msg, 0.0)
    o_ref[0, ...] = msg[:, :128]
    o_ref[1, ...] = msg[:, 128:]


def _tc_edge_mlp(g, e, w1e, b1, w2, b2, block_e):
    _, n_edges, gpack = g.shape
    emb = w2.shape[0]
    edim = e.shape[1]
    grid = n_edges // block_e
    return pl.pallas_call(
        _edge_mlp_body,
        grid=(grid,),
        in_specs=[
            pl.BlockSpec((2, block_e, gpack), lambda i: (0, i, 0)),
            pl.BlockSpec((block_e, edim), lambda i: (i, 0)),
            pl.BlockSpec((edim, emb), lambda i: (0, 0)),
            pl.BlockSpec((1, emb), lambda i: (0, 0)),
            pl.BlockSpec((emb, emb), lambda i: (0, 0)),
            pl.BlockSpec((1, emb), lambda i: (0, 0)),
        ],
        out_specs=pl.BlockSpec((2, block_e, 128), lambda i: (0, i, 0)),
        out_shape=jax.ShapeDtypeStruct((2, n_edges, 128), F32),
    )(g, e, w1e, b1, w2, b2)


def _node_mlp_body(x_ref, *refs):
    (w1x_ref, w1a_ref, b1_ref, w2_ref, b2_ref, wp_ref, o_ref, p_ref) = \
        refs[-8:]
    a_refs = refs[:-8]
    a0 = sum(a[0, ...] for a in a_refs)
    a1 = sum(a[1, ...] for a in a_refs)
    agg = jnp.concatenate([a0, a1], axis=-1).astype(BF16)
    t = (jnp.dot(x_ref[...].astype(BF16), w1x_ref[...],
                 preferred_element_type=F32)
         + jnp.dot(agg, w1a_ref[...], preferred_element_type=F32)
         + b1_ref[...])
    t = jnp.maximum(t, 0.0).astype(BF16)
    y = jnp.dot(t, w2_ref[...], preferred_element_type=F32) + b2_ref[...]
    y = jnp.maximum(y, 0.0)
    o_ref[...] = y
    p = jnp.dot(y.astype(BF16), wp_ref[...], preferred_element_type=F32)
    p_ref[...] = _pack_bf16_pair(p[:, :128], p[:, 128:])


def _tc_node_mlp(x, aggs, w1x, w1a, b1, w2, b2, wp, block_m):
    n, emb = x.shape
    grid = n // block_m
    return pl.pallas_call(
        _node_mlp_body,
        grid=(grid,),
        in_specs=[
            pl.BlockSpec((block_m, emb), lambda i: (i, 0)),
        ] + [
            pl.BlockSpec((2, block_m, 128), lambda i: (0, i, 0))
            for _ in aggs
        ] + [
            pl.BlockSpec((emb, emb), lambda i: (0, 0)),
            pl.BlockSpec((emb, emb), lambda i: (0, 0)),
            pl.BlockSpec((1, emb), lambda i: (0, 0)),
            pl.BlockSpec((emb, emb), lambda i: (0, 0)),
            pl.BlockSpec((1, emb), lambda i: (0, 0)),
            pl.BlockSpec((emb, emb), lambda i: (0, 0)),
        ],
        out_specs=[
            pl.BlockSpec((block_m, emb), lambda i: (i, 0)),
            pl.BlockSpec((block_m, emb // 2), lambda i: (i, 0)),
        ],
        out_shape=[
            jax.ShapeDtypeStruct((n, emb), F32),
            jax.ShapeDtypeStruct((n, emb // 2), F32),
        ],
    )(x, *aggs, w1x, w1a, b1, w2, b2, wp)


def _sc_gather_body(n_edges, pc_hbm, pv_hbm, row_hbm, col_hbm, g_hbm,
                    idxr_all, idxc_all, gb0, gb1, tb0, tb1,
                    sg0, sg1, st0, st1, so0, so1, sp0, sp1):
    cid = lax.axis_index("c")
    sid = lax.axis_index("s")
    w = sid * _NC + cid
    epw = n_edges // _NW
    nfull = epw // _CHUNK
    tail = epw - nfull * _CHUNK
    ebase = w * epw

    pltpu.sync_copy(row_hbm.at[pl.ds(ebase, epw)], idxr_all)
    pltpu.sync_copy(col_hbm.at[pl.ds(ebase, epw)], idxc_all)

    gbufs = (gb0, gb1)
    tbufs = (tb0, tb1)
    sgs = (sg0, sg1)
    sts = (st0, st1)
    sos = (so0, so1)
    sps = (sp0, sp1)

    def start_gathers(k, bo):
        idx_r = idxr_all.at[pl.ds(k * _CHUNK, _CHUNK)]
        idx_c = idxc_all.at[pl.ds(k * _CHUNK, _CHUNK)]
        pltpu.async_copy(pc_hbm.at[idx_r], gbufs[bo], sgs[bo])
        pltpu.async_copy(pv_hbm.at[idx_c], tbufs[bo], sts[bo])

    def wait_gathers(bo):
        pltpu.make_async_copy(
            pc_hbm.at[idxr_all.at[pl.ds(0, _CHUNK)]], gbufs[bo],
            sgs[bo]).wait()
        pltpu.make_async_copy(
            pv_hbm.at[idxc_all.at[pl.ds(0, _CHUNK)]], tbufs[bo],
            sts[bo]).wait()

    def wait_writes(bo):
        pltpu.make_async_copy(
            gbufs[bo], g_hbm.at[0, pl.ds(0, _CHUNK)], sos[bo]).wait()
        pltpu.make_async_copy(
            tbufs[bo], g_hbm.at[1, pl.ds(0, _CHUNK)], sps[bo]).wait()

    start_gathers(0, 0)

    @pl.loop(1, 2 * ((nfull + 2) // 2) + 1, step=2)
    def _(g):
        for bo in range(2):
            k = g + bo
            b = bo ^ 1

            @pl.when(k <= nfull)
            def _():
                @pl.when((k >= 2) & (k < nfull))
                def _():
                    wait_writes(b)

                @pl.when(k < nfull)
                def _():
                    start_gathers(k, b)
                wait_gathers(b ^ 1)
                base = ebase + (k - 1) * _CHUNK
                pltpu.async_copy(gbufs[b ^ 1],
                                 g_hbm.at[0, pl.ds(base, _CHUNK)], sos[b ^ 1])
                pltpu.async_copy(tbufs[b ^ 1],
                                 g_hbm.at[1, pl.ds(base, _CHUNK)], sps[b ^ 1])

    for bo in range(min(2, nfull)):
        wait_writes(bo)

    if tail:
        toff = nfull * _CHUNK
        it_r = idxr_all.at[pl.ds(toff, tail)]
        it_c = idxc_all.at[pl.ds(toff, tail)]
        pltpu.async_copy(pc_hbm.at[it_r], gb0.at[pl.ds(0, tail)], sg0).wait()
        pltpu.async_copy(pv_hbm.at[it_c], tb0.at[pl.ds(0, tail)], st0).wait()
        pltpu.sync_copy(gb0.at[pl.ds(0, tail)],
                        g_hbm.at[0, pl.ds(ebase + toff, tail)])
        pltpu.sync_copy(tb0.at[pl.ds(0, tail)],
                        g_hbm.at[1, pl.ds(ebase + toff, tail)])


def _sc_gather_combine(pc, pv, row, col):
    n_edges = row.shape[0]
    emb = pc.shape[1]
    dt = pc.dtype
    epw = n_edges // _NW
    mesh = plsc.VectorSubcoreMesh(core_axis_name="c", subcore_axis_name="s")
    return pl.kernel(
        functools.partial(_sc_gather_body, n_edges),
        out_type=jax.ShapeDtypeStruct((2, n_edges, emb), dt),
        mesh=mesh,
        scratch_types=[
            pltpu.VMEM((epw,), jnp.int32),
            pltpu.VMEM((epw,), jnp.int32),
            pltpu.VMEM((_CHUNK, emb), dt),
            pltpu.VMEM((_CHUNK, emb), dt),
            pltpu.VMEM((_CHUNK, emb), dt),
            pltpu.VMEM((_CHUNK, emb), dt),
        ] + [pltpu.SemaphoreType.DMA] * 8,
    )(pc, pv, row, col)


def _sc_scatter_body(n_edges, rows_per_tile, msg_hbm, idx_hbm, zero_hbm,
                     agg_hbm, mb0, mb1, ic0, ic1, itail, acc_sh,
                     sm0, sm1, si0, si1):
    cid = lax.axis_index("c")
    sid = lax.axis_index("s")
    rbase = sid * rows_per_tile

    n_zc = rows_per_tile // _CHUNK
    pltpu.sync_copy(zero_hbm, mb0)
    for z in range(n_zc):
        pltpu.sync_copy(mb0, acc_sh.at[pl.ds(rbase + z * _CHUNK, _CHUNK)])
    plsc.subcore_barrier()

    epw = n_edges // _NS
    nfull = epw // _CHUNK
    tail = epw - nfull * _CHUNK
    ebase = sid * epw

    mbs = (mb0, mb1)
    sms = (sm0, sm1)
    icur = (ic0, ic1)
    sis = (si0, si1)
    for bo in range(2):
        pltpu.async_copy(msg_hbm.at[cid, pl.ds(ebase + bo * _CHUNK, _CHUNK)],
                         mbs[bo], sms[bo])
        pltpu.async_copy(idx_hbm.at[pl.ds(ebase + bo * _CHUNK, _CHUNK)],
                         icur[bo], sis[bo])

    @pl.loop(0, nfull, step=2)
    def _(g):
        for bo in range(2):
            k = g + bo
            pltpu.make_async_copy(
                msg_hbm.at[cid, pl.ds(0, _CHUNK)], mbs[bo], sms[bo]).wait()
            pltpu.make_async_copy(
                idx_hbm.at[pl.ds(0, _CHUNK)], icur[bo], sis[bo]).wait()
            pltpu.sync_copy(mbs[bo], acc_sh.at[icur[bo]], add=True)

            @pl.when(k + 2 < nfull)
            def _():
                nb = ebase + (k + 2) * _CHUNK
                pltpu.async_copy(
                    msg_hbm.at[cid, pl.ds(nb, _CHUNK)], mbs[bo], sms[bo])
                pltpu.async_copy(
                    idx_hbm.at[pl.ds(nb, _CHUNK)], icur[bo], sis[bo])

    if tail:
        toff = nfull * _CHUNK
        cm = pltpu.async_copy(msg_hbm.at[cid, pl.ds(ebase + toff, tail)],
                              mb0.at[pl.ds(0, tail)], sm0)
        ci = pltpu.async_copy(idx_hbm.at[pl.ds(ebase + toff, tail)],
                              itail, si0)
        cm.wait()
        ci.wait()
        pltpu.sync_copy(mb0.at[pl.ds(0, tail)], acc_sh.at[itail], add=True)
    plsc.subcore_barrier()

    for z in range(n_zc):
        pltpu.sync_copy(acc_sh.at[pl.ds(rbase + z * _CHUNK, _CHUNK)], mb0)
        pltpu.sync_copy(mb0, agg_hbm.at[cid, pl.ds(rbase + z * _CHUNK, _CHUNK)])


def _sc_scatter_add(msg2, idx, n_nodes):
    n_edges = idx.shape[0]
    half = msg2.shape[2]
    rows_per_tile = (-(-n_nodes // _NS) + _CHUNK - 1) // _CHUNK * _CHUNK
    n_pad = rows_per_tile * _NS
    zero = jnp.zeros((_CHUNK, half), F32)
    mesh = plsc.VectorSubcoreMesh(core_axis_name="c", subcore_axis_name="s")
    epw = n_edges // _NS
    tail = epw - (epw // _CHUNK) * _CHUNK
    return pl.kernel(
        functools.partial(_sc_scatter_body, n_edges, rows_per_tile),
        out_type=jax.ShapeDtypeStruct((2, n_pad, half), F32),
        mesh=mesh,
        scratch_types=[
            pltpu.VMEM((_CHUNK, half), F32),
            pltpu.VMEM((_CHUNK, half), F32),
            pltpu.VMEM((_CHUNK,), jnp.int32),
            pltpu.VMEM((_CHUNK,), jnp.int32),
            pltpu.VMEM((tail or _CHUNK,), jnp.int32),
            pltpu.VMEM_SHARED((n_pad, half), F32),
            pltpu.SemaphoreType.DMA,
            pltpu.SemaphoreType.DMA,
            pltpu.SemaphoreType.DMA,
            pltpu.SemaphoreType.DMA,
        ],
    )(msg2, idx, zero)


def kernel(c, v, edge_index, e,
           gC_W1, gC_b1, gC_W2, gC_b2,
           gV_W1, gV_b1, gV_W2, gV_b2,
           fC_W1, fC_b1, fC_W2, fC_b2,
           fV_W1, fV_b1, fV_W2, fV_b2,
           beta_c, sigma_c, beta_v, sigma_v):
    n_c, emb = c.shape
    n_v = v.shape[0]
    row = edge_index[0]
    col = edge_index[1]

    gC_W1c, gC_W1v, gC_W1e = gC_W1[:emb], gC_W1[emb:2 * emb], gC_W1[2 * emb:]
    gV_W1c, gV_W1v, gV_W1e = gV_W1[:emb], gV_W1[emb:2 * emb], gV_W1[2 * emb:]

    fC_W1x, fC_W1a = fC_W1[:emb], fC_W1[emb:]
    fV_W1x, fV_W1a = fV_W1[:emb], fV_W1[emb:]
    fC_W1a_eff = fC_W1a / sigma_c[:, None]
    fC_b1_eff = fC_b1 - (beta_c / sigma_c) @ fC_W1a
    fV_W1a_eff = fV_W1a / sigma_v[:, None]
    fV_b1_eff = fV_b1 - (beta_v / sigma_v) @ fV_W1a

    b2 = lambda x: x.reshape(1, -1)
    bf = lambda x: x.astype(BF16)

    pc1 = _tc_matmul(c, gC_W1c, 5000, pack=True)
    pv_both = _tc_matmul(v, jnp.concatenate([gC_W1v, gV_W1v], axis=1), 5000,
                         pack=True)
    pv1 = pv_both[:, :emb // 2]
    pv2 = pv_both[:, emb // 2:]

    n_edges = row.shape[0]
    quantum = 32 * _CHUNK
    n_parts = 2
    cut = [min((n_edges * i // n_parts + quantum - 1) // quantum * quantum,
               n_edges) for i in range(n_parts + 1)]
    halves = [(row[a:b], col[a:b], e[a:b])
              for a, b in zip(cut[:-1], cut[1:]) if b > a]

    def phase(pc, pv, gW1e, gb1, gW2, gb2, dst, nn):
        msgs = []
        for rh, ch, eh in halves:
            gh = _sc_gather_combine(pc, pv, rh, ch)
            bl = rh.shape[0] // 32
            msgs.append(_tc_edge_mlp(gh, eh, gW1e, b2(gb1), bf(gW2),
                                     b2(gb2), bl))
        aggs = [_sc_scatter_add(m, (rh if dst == 0 else ch), nn)
                for m, (rh, ch, _) in zip(msgs, halves)]
        return aggs

    agg_c = phase(pc1, pv1, gC_W1e, gC_b1, gC_W2, gC_b2, 0, n_c)
    c_new, pc2 = _tc_node_mlp(c, agg_c, bf(fC_W1x),
                              bf(fC_W1a_eff), b2(fC_b1_eff), bf(fC_W2),
                              b2(fC_b2), bf(gV_W1c), 5000)

    agg_v = phase(pc2, pv2, gV_W1e, gV_b1, gV_W2, gV_b2, 1, n_v)
    v_new, _ = _tc_node_mlp(v, agg_v, bf(fV_W1x),
                            bf(fV_W1a_eff), b2(fV_b1_eff), bf(fV_W2),
                            b2(fV_b2), bf(fV_W2), 5000)

    return (c_new, v_new)


# speedup vs baseline: 1.0466x; 1.0088x over previous
import functools

import jax
import jax.numpy as jnp
from jax import lax
from jax.experimental import pallas as pl
from jax.experimental.pallas import tpu as pltpu
from jax.experimental.pallas import tpu_sc as plsc

F32 = jnp.float32
BF16 = jnp.bfloat16

_NC = 2
_NS = 16
_NW = _NC * _NS
_CHUNK = 128


def _pack_bf16_pair(a, b):
    au = lax.bitcast_convert_type(a.astype(BF16), jnp.uint16)
    bu = lax.bitcast_convert_type(b.astype(BF16), jnp.uint16)
    word = au.astype(jnp.uint32) | (bu.astype(jnp.uint32) << 16)
    return lax.bitcast_convert_type(word, F32)


def _unpack_bf16_pair(w):
    u = lax.bitcast_convert_type(w, jnp.uint32)
    a = lax.bitcast_convert_type((u & 0xFFFF).astype(jnp.uint16), BF16)
    b = lax.bitcast_convert_type((u >> 16).astype(jnp.uint16), BF16)
    return a, b


def _mm_body(x_ref, w_ref, o_ref):
    y = jnp.dot(x_ref[...], w_ref[...], preferred_element_type=F32)
    if o_ref.shape[1] * 2 == y.shape[1]:
        parts = []
        for gi in range(y.shape[1] // 256):
            parts.append(_pack_bf16_pair(y[:, gi * 256:gi * 256 + 128],
                                         y[:, gi * 256 + 128:(gi + 1) * 256]))
        y = jnp.concatenate(parts, axis=-1) if len(parts) > 1 else parts[0]
    o_ref[...] = y.astype(o_ref.dtype)


def _tc_matmul(x, w, block_m, pack=False):
    m, k = x.shape
    _, n = w.shape
    n_out = n // 2 if pack else n
    grid = m // block_m
    return pl.pallas_call(
        _mm_body,
        grid=(grid,),
        in_specs=[
            pl.BlockSpec((block_m, k), lambda i: (i, 0)),
            pl.BlockSpec((k, n), lambda i: (0, 0)),
        ],
        out_specs=pl.BlockSpec((block_m, n_out), lambda i: (i, 0)),
        out_shape=jax.ShapeDtypeStruct((m, n_out), F32),
    )(x, w)


def _edge_mlp_body(g_ref, e_ref, w1e_ref, b1_ref, w2_ref, b2_ref, o_ref):
    a0, b0 = _unpack_bf16_pair(g_ref[0, ...])
    a1, b1h = _unpack_bf16_pair(g_ref[1, ...])
    gsum = (jnp.concatenate([a0, b0], axis=-1).astype(F32)
            + jnp.concatenate([a1, b1h], axis=-1).astype(F32))
    h = (gsum
         + jnp.dot(e_ref[...], w1e_ref[...], preferred_element_type=F32)
         + b1_ref[...])
    h = jnp.maximum(h, 0.0).astype(BF16)
    msg = jnp.dot(h, w2_ref[...], preferred_element_type=F32) + b2_ref[...]
    msg = jnp.maximum(msg, 0.0)
    o_ref[0, ...] = msg[:, :128]
    o_ref[1, ...] = msg[:, 128:]


def _tc_edge_mlp(g, e, w1e, b1, w2, b2, block_e):
    _, n_edges, gpack = g.shape
    emb = w2.shape[0]
    edim = e.shape[1]
    grid = n_edges // block_e
    return pl.pallas_call(
        _edge_mlp_body,
        grid=(grid,),
        in_specs=[
            pl.BlockSpec((2, block_e, gpack), lambda i: (0, i, 0)),
            pl.BlockSpec((block_e, edim), lambda i: (i, 0)),
            pl.BlockSpec((edim, emb), lambda i: (0, 0)),
            pl.BlockSpec((1, emb), lambda i: (0, 0)),
            pl.BlockSpec((emb, emb), lambda i: (0, 0)),
            pl.BlockSpec((1, emb), lambda i: (0, 0)),
        ],
        out_specs=pl.BlockSpec((2, block_e, 128), lambda i: (0, i, 0)),
        out_shape=jax.ShapeDtypeStruct((2, n_edges, 128), F32),
    )(g, e, w1e, b1, w2, b2)


def _node_mlp_body(x_ref, *refs):
    (w1x_ref, w1a_ref, b1_ref, w2_ref, b2_ref, wp_ref, o_ref, p_ref) = \
        refs[-8:]
    a_refs = refs[:-8]
    a0 = sum(a[0, ...] for a in a_refs)
    a1 = sum(a[1, ...] for a in a_refs)
    agg = jnp.concatenate([a0, a1], axis=-1).astype(BF16)
    t = (jnp.dot(x_ref[...].astype(BF16), w1x_ref[...],
                 preferred_element_type=F32)
         + jnp.dot(agg, w1a_ref[...], preferred_element_type=F32)
         + b1_ref[...])
    t = jnp.maximum(t, 0.0).astype(BF16)
    y = jnp.dot(t, w2_ref[...], preferred_element_type=F32) + b2_ref[...]
    y = jnp.maximum(y, 0.0)
    o_ref[...] = y
    p = jnp.dot(y.astype(BF16), wp_ref[...], preferred_element_type=F32)
    p_ref[...] = _pack_bf16_pair(p[:, :128], p[:, 128:])


def _tc_node_mlp(x, aggs, w1x, w1a, b1, w2, b2, wp, block_m):
    n, emb = x.shape
    grid = n // block_m
    return pl.pallas_call(
        _node_mlp_body,
        grid=(grid,),
        in_specs=[
            pl.BlockSpec((block_m, emb), lambda i: (i, 0)),
        ] + [
            pl.BlockSpec((2, block_m, 128), lambda i: (0, i, 0))
            for _ in aggs
        ] + [
            pl.BlockSpec((emb, emb), lambda i: (0, 0)),
            pl.BlockSpec((emb, emb), lambda i: (0, 0)),
            pl.BlockSpec((1, emb), lambda i: (0, 0)),
            pl.BlockSpec((emb, emb), lambda i: (0, 0)),
            pl.BlockSpec((1, emb), lambda i: (0, 0)),
            pl.BlockSpec((emb, emb), lambda i: (0, 0)),
        ],
        out_specs=[
            pl.BlockSpec((block_m, emb), lambda i: (i, 0)),
            pl.BlockSpec((block_m, emb // 2), lambda i: (i, 0)),
        ],
        out_shape=[
            jax.ShapeDtypeStruct((n, emb), F32),
            jax.ShapeDtypeStruct((n, emb // 2), F32),
        ],
    )(x, *aggs, w1x, w1a, b1, w2, b2, wp)


def _sc_gather_body(n_edges, pc_hbm, pv_hbm, row_hbm, col_hbm, g_hbm,
                    idxr_all, idxc_all, gb0, gb1, tb0, tb1,
                    sg0, sg1, st0, st1, so0, so1, sp0, sp1):
    cid = lax.axis_index("c")
    sid = lax.axis_index("s")
    w = sid * _NC + cid
    epw = n_edges // _NW
    nfull = epw // _CHUNK
    tail = epw - nfull * _CHUNK
    ebase = w * epw

    pltpu.sync_copy(row_hbm.at[pl.ds(ebase, epw)], idxr_all)
    pltpu.sync_copy(col_hbm.at[pl.ds(ebase, epw)], idxc_all)

    gbufs = (gb0, gb1)
    tbufs = (tb0, tb1)
    sgs = (sg0, sg1)
    sts = (st0, st1)
    sos = (so0, so1)
    sps = (sp0, sp1)

    def start_gathers(k, bo):
        idx_r = idxr_all.at[pl.ds(k * _CHUNK, _CHUNK)]
        idx_c = idxc_all.at[pl.ds(k * _CHUNK, _CHUNK)]
        pltpu.async_copy(pc_hbm.at[idx_r], gbufs[bo], sgs[bo])
        pltpu.async_copy(pv_hbm.at[idx_c], tbufs[bo], sts[bo])

    def wait_gathers(bo):
        pltpu.make_async_copy(
            pc_hbm.at[idxr_all.at[pl.ds(0, _CHUNK)]], gbufs[bo],
            sgs[bo]).wait()
        pltpu.make_async_copy(
            pv_hbm.at[idxc_all.at[pl.ds(0, _CHUNK)]], tbufs[bo],
            sts[bo]).wait()

    def wait_writes(bo):
        pltpu.make_async_copy(
            gbufs[bo], g_hbm.at[0, pl.ds(0, _CHUNK)], sos[bo]).wait()
        pltpu.make_async_copy(
            tbufs[bo], g_hbm.at[1, pl.ds(0, _CHUNK)], sps[bo]).wait()

    start_gathers(0, 0)

    @pl.loop(1, 2 * ((nfull + 2) // 2) + 1, step=2)
    def _(g):
        for bo in range(2):
            k = g + bo
            b = bo ^ 1

            @pl.when(k <= nfull)
            def _():
                @pl.when((k >= 2) & (k < nfull))
                def _():
                    wait_writes(b)

                @pl.when(k < nfull)
                def _():
                    start_gathers(k, b)
                wait_gathers(b ^ 1)
                base = ebase + (k - 1) * _CHUNK
                pltpu.async_copy(gbufs[b ^ 1],
                                 g_hbm.at[0, pl.ds(base, _CHUNK)], sos[b ^ 1])
                pltpu.async_copy(tbufs[b ^ 1],
                                 g_hbm.at[1, pl.ds(base, _CHUNK)], sps[b ^ 1])

    for bo in range(min(2, nfull)):
        wait_writes(bo)

    if tail:
        toff = nfull * _CHUNK
        it_r = idxr_all.at[pl.ds(toff, tail)]
        it_c = idxc_all.at[pl.ds(toff, tail)]
        pltpu.async_copy(pc_hbm.at[it_r], gb0.at[pl.ds(0, tail)], sg0).wait()
        pltpu.async_copy(pv_hbm.at[it_c], tb0.at[pl.ds(0, tail)], st0).wait()
        pltpu.sync_copy(gb0.at[pl.ds(0, tail)],
                        g_hbm.at[0, pl.ds(ebase + toff, tail)])
        pltpu.sync_copy(tb0.at[pl.ds(0, tail)],
                        g_hbm.at[1, pl.ds(ebase + toff, tail)])


def _sc_gather_combine(pc, pv, row, col):
    n_edges = row.shape[0]
    emb = pc.shape[1]
    dt = pc.dtype
    epw = n_edges // _NW
    mesh = plsc.VectorSubcoreMesh(core_axis_name="c", subcore_axis_name="s")
    return pl.kernel(
        functools.partial(_sc_gather_body, n_edges),
        out_type=jax.ShapeDtypeStruct((2, n_edges, emb), dt),
        mesh=mesh,
        scratch_types=[
            pltpu.VMEM((epw,), jnp.int32),
            pltpu.VMEM((epw,), jnp.int32),
            pltpu.VMEM((_CHUNK, emb), dt),
            pltpu.VMEM((_CHUNK, emb), dt),
            pltpu.VMEM((_CHUNK, emb), dt),
            pltpu.VMEM((_CHUNK, emb), dt),
        ] + [pltpu.SemaphoreType.DMA] * 8,
    )(pc, pv, row, col)


def _sc_scatter_body(n_edges, rows_per_tile, msg_hbm, idx_hbm, zero_hbm,
                     agg_hbm, mb0, mb1, ic0, ic1, itail, acc_sh,
                     sm0, sm1, si0, si1):
    cid = lax.axis_index("c")
    sid = lax.axis_index("s")
    rbase = sid * rows_per_tile

    n_zc = rows_per_tile // _CHUNK
    pltpu.sync_copy(zero_hbm, mb0)
    for z in range(n_zc):
        pltpu.sync_copy(mb0, acc_sh.at[pl.ds(rbase + z * _CHUNK, _CHUNK)])
    plsc.subcore_barrier()

    epw = n_edges // _NS
    nfull = epw // _CHUNK
    tail = epw - nfull * _CHUNK
    ebase = sid * epw

    mbs = (mb0, mb1)
    sms = (sm0, sm1)
    icur = (ic0, ic1)
    sis = (si0, si1)
    for bo in range(2):
        pltpu.async_copy(msg_hbm.at[cid, pl.ds(ebase + bo * _CHUNK, _CHUNK)],
                         mbs[bo], sms[bo])
        pltpu.async_copy(idx_hbm.at[pl.ds(ebase + bo * _CHUNK, _CHUNK)],
                         icur[bo], sis[bo])

    @pl.loop(0, nfull, step=2)
    def _(g):
        for bo in range(2):
            k = g + bo
            pltpu.make_async_copy(
                msg_hbm.at[cid, pl.ds(0, _CHUNK)], mbs[bo], sms[bo]).wait()
            pltpu.make_async_copy(
                idx_hbm.at[pl.ds(0, _CHUNK)], icur[bo], sis[bo]).wait()
            pltpu.sync_copy(mbs[bo], acc_sh.at[icur[bo]], add=True)

            @pl.when(k + 2 < nfull)
            def _():
                nb = ebase + (k + 2) * _CHUNK
                pltpu.async_copy(
                    msg_hbm.at[cid, pl.ds(nb, _CHUNK)], mbs[bo], sms[bo])
                pltpu.async_copy(
                    idx_hbm.at[pl.ds(nb, _CHUNK)], icur[bo], sis[bo])

    if tail:
        toff = nfull * _CHUNK
        cm = pltpu.async_copy(msg_hbm.at[cid, pl.ds(ebase + toff, tail)],
                              mb0.at[pl.ds(0, tail)], sm0)
        ci = pltpu.async_copy(idx_hbm.at[pl.ds(ebase + toff, tail)],
                              itail, si0)
        cm.wait()
        ci.wait()
        pltpu.sync_copy(mb0.at[pl.ds(0, tail)], acc_sh.at[itail], add=True)
    plsc.subcore_barrier()

    for z in range(n_zc):
        pltpu.sync_copy(acc_sh.at[pl.ds(rbase + z * _CHUNK, _CHUNK)], mb0)
        pltpu.sync_copy(mb0, agg_hbm.at[cid, pl.ds(rbase + z * _CHUNK, _CHUNK)])


def _sc_scatter_add(msg2, idx, n_nodes):
    n_edges = idx.shape[0]
    half = msg2.shape[2]
    rows_per_tile = (-(-n_nodes // _NS) + _CHUNK - 1) // _CHUNK * _CHUNK
    n_pad = rows_per_tile * _NS
    zero = jnp.zeros((_CHUNK, half), F32)
    mesh = plsc.VectorSubcoreMesh(core_axis_name="c", subcore_axis_name="s")
    epw = n_edges // _NS
    tail = epw - (epw // _CHUNK) * _CHUNK
    return pl.kernel(
        functools.partial(_sc_scatter_body, n_edges, rows_per_tile),
        out_type=jax.ShapeDtypeStruct((2, n_pad, half), F32),
        mesh=mesh,
        scratch_types=[
            pltpu.VMEM((_CHUNK, half), F32),
            pltpu.VMEM((_CHUNK, half), F32),
            pltpu.VMEM((_CHUNK,), jnp.int32),
            pltpu.VMEM((_CHUNK,), jnp.int32),
            pltpu.VMEM((tail or _CHUNK,), jnp.int32),
            pltpu.VMEM_SHARED((n_pad, half), F32),
            pltpu.SemaphoreType.DMA,
            pltpu.SemaphoreType.DMA,
            pltpu.SemaphoreType.DMA,
            pltpu.SemaphoreType.DMA,
        ],
    )(msg2, idx, zero)


def kernel(c, v, edge_index, e,
           gC_W1, gC_b1, gC_W2, gC_b2,
           gV_W1, gV_b1, gV_W2, gV_b2,
           fC_W1, fC_b1, fC_W2, fC_b2,
           fV_W1, fV_b1, fV_W2, fV_b2,
           beta_c, sigma_c, beta_v, sigma_v):
    n_c, emb = c.shape
    n_v = v.shape[0]
    row = edge_index[0]
    col = edge_index[1]

    gC_W1c, gC_W1v, gC_W1e = gC_W1[:emb], gC_W1[emb:2 * emb], gC_W1[2 * emb:]
    gV_W1c, gV_W1v, gV_W1e = gV_W1[:emb], gV_W1[emb:2 * emb], gV_W1[2 * emb:]

    fC_W1x, fC_W1a = fC_W1[:emb], fC_W1[emb:]
    fV_W1x, fV_W1a = fV_W1[:emb], fV_W1[emb:]
    fC_W1a_eff = fC_W1a / sigma_c[:, None]
    fC_b1_eff = fC_b1 - (beta_c / sigma_c) @ fC_W1a
    fV_W1a_eff = fV_W1a / sigma_v[:, None]
    fV_b1_eff = fV_b1 - (beta_v / sigma_v) @ fV_W1a

    b2 = lambda x: x.reshape(1, -1)
    bf = lambda x: x.astype(BF16)

    pc1 = _tc_matmul(c, gC_W1c, 5000, pack=True)
    pv_both = _tc_matmul(v, jnp.concatenate([gC_W1v, gV_W1v], axis=1), 5000,
                         pack=True)
    pv1 = pv_both[:, :emb // 2]
    pv2 = pv_both[:, emb // 2:]

    n_edges = row.shape[0]
    quantum = 32 * _CHUNK
    n_parts = 2
    cut = [min((n_edges * i // n_parts + quantum - 1) // quantum * quantum,
               n_edges) for i in range(n_parts + 1)]
    halves = [(row[a:b], col[a:b], e[a:b])
              for a, b in zip(cut[:-1], cut[1:]) if b > a]

    def phase(pc, pv, gW1e, gb1, gW2, gb2, dst, nn):
        msgs = []
        for rh, ch, eh in halves:
            gh = _sc_gather_combine(pc, pv, rh, ch)
            bl = rh.shape[0] // 16
            msgs.append(_tc_edge_mlp(gh, eh, gW1e, b2(gb1), bf(gW2),
                                     b2(gb2), bl))
        aggs = [_sc_scatter_add(m, (rh if dst == 0 else ch), nn)
                for m, (rh, ch, _) in zip(msgs, halves)]
        return aggs

    agg_c = phase(pc1, pv1, gC_W1e, gC_b1, gC_W2, gC_b2, 0, n_c)
    c_new, pc2 = _tc_node_mlp(c, agg_c, bf(fC_W1x),
                              bf(fC_W1a_eff), b2(fC_b1_eff), bf(fC_W2),
                              b2(fC_b2), bf(gV_W1c), 5000)

    agg_v = phase(pc2, pv2, gV_W1e, gV_b1, gV_W2, gV_b2, 1, n_v)
    v_new, _ = _tc_node_mlp(v, agg_v, bf(fV_W1x),
                            bf(fV_W1a_eff), b2(fV_b1_eff), bf(fV_W2),
                            b2(fV_b2), bf(fV_W2), 5000)

    return (c_new, v_new)
